# baseline (speedup 1.0000x reference)
import jax
import jax.numpy as jnp
from jax import lax
from jax.experimental import pallas as pl
from jax.experimental.pallas import tpu as pltpu

VOCAB = 100000
EBD = 64
NCTX = 8
HID = 128
NOPS = 8
BV = 512
SPAN = NOPS * BV
NJ = (VOCAB + SPAN - 1) // SPAN
NBLK = (VOCAB + BV - 1) // BV
NROW = NJ * NOPS


def _mlp_body(*refs):
    (idx_ref, ebdt_ref, w1_ref, b1_ref) = refs[:4]
    w2s = refs[4:4 + NOPS]
    (b2_ref, out_ref,
     x_ref, win_ref, h_ref, logit_ref, m_ref, s_ref, sem) = refs[4 + NOPS:]
    p = pl.program_id(0)
    j = pl.program_id(1)

    @pl.when((p == 0) & (j == 0))
    def _():
        copies = []
        for i in range(NCTX):
            base = pl.multiple_of((idx_ref[i] // 128) * 128, 128)
            c = pltpu.make_async_copy(
                ebdt_ref.at[:, pl.ds(base, 128)], win_ref.at[i], sem)
            c.start()
            copies.append(c)
        for c in copies:
            c.wait()
        lane = lax.broadcasted_iota(jnp.int32, (EBD, 128), 1)
        for i in range(NCTX):
            off = idx_ref[i] % 128
            sel = jnp.where(lane == off, win_ref[i], 0.0)
            x_ref[pl.ds(i * EBD, EBD), :] = jnp.sum(sel, axis=1,
                                                    keepdims=True)
        h = lax.dot_general(w1_ref[...], x_ref[...],
                            (((1,), (0,)), ((), ())),
                            preferred_element_type=jnp.float32)
        h_ref[...] = jnp.maximum(h + b1_ref[...][:, None], 0.0)
        m_ref[0] = -jnp.inf
        s_ref[0] = 0.0

    @pl.when(p == 0)
    def _():
        for k, w2_k in enumerate(w2s):
            logits = lax.dot_general(h_ref[...], w2_k[...],
                                     (((0,), (1,)), ((), ())),
                                     preferred_element_type=jnp.float32)
            logits = logits + b2_ref[pl.ds(k * BV, BV)][None, :]
            col = (j * SPAN + k * BV
                   + lax.broadcasted_iota(jnp.int32, (1, BV), 1))
            logits = jnp.where(col < VOCAB, logits, -jnp.inf)
            logit_ref[pl.ds(j * NOPS + k, 1), :] = logits
            bm = jnp.max(logits)
            m_old = m_ref[0]
            m_new = jnp.maximum(m_old, bm)
            s_ref[0] = s_ref[0] * jnp.exp(m_old - m_new) + jnp.sum(
                jnp.exp(logits - m_new))
            m_ref[0] = m_new

    @pl.when(p == 1)
    def _():
        lse = m_ref[0] + jnp.log(s_ref[0])
        for k in range(NOPS):
            out_ref[:, k * BV:(k + 1) * BV] = (
                logit_ref[pl.ds(j * NOPS + k, 1), :] - lse)


def _w2_spec(k):
    return pl.BlockSpec(
        (BV, HID),
        lambda p, j: (jnp.where((p == 0) & (j < NJ),
                                jnp.minimum(NOPS * j + k, NBLK - 1),
                                NBLK - 1), 0))


def kernel(inputs, ebd, W1, b1, W2, b2):
    idx = inputs.astype(jnp.int32)
    ebdt = ebd.T
    return pl.pallas_call(
        _mlp_body,
        grid=(2, NJ),
        in_specs=[
            pl.BlockSpec(memory_space=pltpu.SMEM),
            pl.BlockSpec(memory_space=pl.ANY),
            pl.BlockSpec((HID, NCTX * EBD), lambda p, j: (0, 0)),
            pl.BlockSpec((HID,), lambda p, j: (0,)),
            *[_w2_spec(k) for k in range(NOPS)],
            pl.BlockSpec((SPAN,), lambda p, j: (jnp.where(p == 0, j, NJ - 1),)),
        ],
        out_specs=pl.BlockSpec((1, SPAN),
                               lambda p, j: (0, jnp.where(p == 0, 0, j))),
        out_shape=jax.ShapeDtypeStruct((1, VOCAB), jnp.float32),
        scratch_shapes=[
            pltpu.VMEM((NCTX * EBD, 1), jnp.float32),
            pltpu.VMEM((NCTX, EBD, 128), jnp.float32),
            pltpu.VMEM((HID, 1), jnp.float32),
            pltpu.VMEM((NROW, BV), jnp.float32),
            pltpu.SMEM((1,), jnp.float32),
            pltpu.SMEM((1,), jnp.float32),
            pltpu.SemaphoreType.DMA,
        ],
        compiler_params=pltpu.CompilerParams(disable_bounds_checks=True),
    )(idx, ebdt, W1, b1, *([W2] * NOPS), b2)


# speedup vs baseline: 1.7340x; 1.7340x over previous
import jax
import jax.numpy as jnp
from jax import lax
from jax.experimental import pallas as pl
from jax.experimental.pallas import tpu as pltpu

VOCAB = 100000
EBD = 64
NCTX = 8
HID = 128
NOPS = 4
BV = 2048
SPAN = NOPS * BV
NJ = (VOCAB + SPAN - 1) // SPAN
NBLK = (VOCAB + BV - 1) // BV
NROW = NJ * NOPS


def _mlp_body(*refs):
    (idx_ref, ebdt_ref, w1_ref, b1_ref) = refs[:4]
    w2s = refs[4:4 + NOPS]
    (b2_ref, out_ref,
     x_ref, win_ref, h_ref, logit_ref, m_ref, s_ref, sem) = refs[4 + NOPS:]
    p = pl.program_id(0)
    j = pl.program_id(1)

    @pl.when((p == 0) & (j == 0))
    def _():
        copies = []
        for i in range(NCTX):
            base = pl.multiple_of((idx_ref[i] // 128) * 128, 128)
            c = pltpu.make_async_copy(
                ebdt_ref.at[:, pl.ds(base, 128)], win_ref.at[i], sem)
            c.start()
            copies.append(c)
        for c in copies:
            c.wait()
        lane = lax.broadcasted_iota(jnp.int32, (EBD, 128), 1)
        for i in range(NCTX):
            off = idx_ref[i] % 128
            sel = jnp.where(lane == off, win_ref[i], 0.0)
            x_ref[pl.ds(i * EBD, EBD), :] = jnp.sum(sel, axis=1,
                                                    keepdims=True)
        h = lax.dot_general(w1_ref[...], x_ref[...],
                            (((1,), (0,)), ((), ())),
                            preferred_element_type=jnp.float32)
        h_ref[...] = jnp.maximum(h + b1_ref[...][:, None], 0.0)
        m_ref[0] = -jnp.inf
        s_ref[0] = 0.0

    @pl.when(p == 0)
    def _():
        for k, w2_k in enumerate(w2s):
            logits = lax.dot_general(h_ref[...], w2_k[...],
                                     (((0,), (1,)), ((), ())),
                                     preferred_element_type=jnp.float32)
            logits = logits + b2_ref[pl.ds(k * BV, BV)][None, :]
            col = (j * SPAN + k * BV
                   + lax.broadcasted_iota(jnp.int32, (1, BV), 1))
            logits = jnp.where(col < VOCAB, logits, -jnp.inf)
            logit_ref[pl.ds(j * NOPS + k, 1), :] = logits
            bm = jnp.max(logits)
            m_old = m_ref[0]
            m_new = jnp.maximum(m_old, bm)
            s_ref[0] = s_ref[0] * jnp.exp(m_old - m_new) + jnp.sum(
                jnp.exp(logits - m_new))
            m_ref[0] = m_new

    @pl.when(p == 1)
    def _():
        lse = m_ref[0] + jnp.log(s_ref[0])
        for k in range(NOPS):
            out_ref[:, k * BV:(k + 1) * BV] = (
                logit_ref[pl.ds(j * NOPS + k, 1), :] - lse)


def _w2_spec(k):
    return pl.BlockSpec(
        (BV, HID),
        lambda p, j: (jnp.where((p == 0) & (j < NJ),
                                jnp.minimum(NOPS * j + k, NBLK - 1),
                                NBLK - 1), 0))


def kernel(inputs, ebd, W1, b1, W2, b2):
    idx = inputs.astype(jnp.int32)
    ebdt = ebd.T
    return pl.pallas_call(
        _mlp_body,
        grid=(2, NJ),
        in_specs=[
            pl.BlockSpec(memory_space=pltpu.SMEM),
            pl.BlockSpec(memory_space=pl.ANY),
            pl.BlockSpec((HID, NCTX * EBD), lambda p, j: (0, 0)),
            pl.BlockSpec((HID,), lambda p, j: (0,)),
            *[_w2_spec(k) for k in range(NOPS)],
            pl.BlockSpec((SPAN,), lambda p, j: (jnp.where(p == 0, j, NJ - 1),)),
        ],
        out_specs=pl.BlockSpec((1, SPAN),
                               lambda p, j: (0, jnp.where(p == 0, 0, j))),
        out_shape=jax.ShapeDtypeStruct((1, VOCAB), jnp.float32),
        scratch_shapes=[
            pltpu.VMEM((NCTX * EBD, 1), jnp.float32),
            pltpu.VMEM((NCTX, EBD, 128), jnp.float32),
            pltpu.VMEM((HID, 1), jnp.float32),
            pltpu.VMEM((NROW, BV), jnp.float32),
            pltpu.SMEM((1,), jnp.float32),
            pltpu.SMEM((1,), jnp.float32),
            pltpu.SemaphoreType.DMA,
        ],
        compiler_params=pltpu.CompilerParams(disable_bounds_checks=True),
    )(idx, ebdt, W1, b1, *([W2] * NOPS), b2)


# speedup vs baseline: 2.0948x; 1.2081x over previous
import jax
import jax.numpy as jnp
from jax import lax
from jax.experimental import pallas as pl
from jax.experimental.pallas import tpu as pltpu

VOCAB = 100000
EBD = 64
NCTX = 8
HID = 128
NOPS = 4
BV = 4096
SPAN = NOPS * BV
NJ = (VOCAB + SPAN - 1) // SPAN
NBLK = (VOCAB + BV - 1) // BV
NROW = NJ * NOPS


def _mlp_body(*refs):
    (idx_ref, ebdt_ref, w1_ref, b1_ref) = refs[:4]
    w2s = refs[4:4 + NOPS]
    (b2_ref, out_ref,
     x_ref, win_ref, h_ref, logit_ref, m_ref, s_ref, sem) = refs[4 + NOPS:]
    p = pl.program_id(0)
    j = pl.program_id(1)

    @pl.when((p == 0) & (j == 0))
    def _():
        copies = []
        for i in range(NCTX):
            base = pl.multiple_of((idx_ref[i] // 128) * 128, 128)
            c = pltpu.make_async_copy(
                ebdt_ref.at[:, pl.ds(base, 128)], win_ref.at[i], sem)
            c.start()
            copies.append(c)
        for c in copies:
            c.wait()
        lane = lax.broadcasted_iota(jnp.int32, (EBD, 128), 1)
        for i in range(NCTX):
            off = idx_ref[i] % 128
            sel = jnp.where(lane == off, win_ref[i], 0.0)
            x_ref[pl.ds(i * EBD, EBD), :] = jnp.sum(sel, axis=1,
                                                    keepdims=True)
        h = lax.dot_general(w1_ref[...], x_ref[...],
                            (((1,), (0,)), ((), ())),
                            preferred_element_type=jnp.float32)
        h_ref[...] = jnp.maximum(h + b1_ref[...][:, None], 0.0)
        m_ref[0] = -jnp.inf
        s_ref[0] = 0.0

    @pl.when(p == 0)
    def _():
        for k, w2_k in enumerate(w2s):
            logits = lax.dot_general(h_ref[...], w2_k[...],
                                     (((0,), (1,)), ((), ())),
                                     preferred_element_type=jnp.float32)
            logits = logits + b2_ref[pl.ds(k * BV, BV)][None, :]
            col = (j * SPAN + k * BV
                   + lax.broadcasted_iota(jnp.int32, (1, BV), 1))
            logits = jnp.where(col < VOCAB, logits, -jnp.inf)
            logit_ref[pl.ds(j * NOPS + k, 1), :] = logits
            bm = jnp.max(logits)
            m_old = m_ref[0]
            m_new = jnp.maximum(m_old, bm)
            s_ref[0] = s_ref[0] * jnp.exp(m_old - m_new) + jnp.sum(
                jnp.exp(logits - m_new))
            m_ref[0] = m_new

    @pl.when(p == 1)
    def _():
        lse = m_ref[0] + jnp.log(s_ref[0])
        for k in range(NOPS):
            out_ref[:, k * BV:(k + 1) * BV] = (
                logit_ref[pl.ds(j * NOPS + k, 1), :] - lse)


def _w2_spec(k):
    return pl.BlockSpec(
        (BV, HID),
        lambda p, j: (jnp.where((p == 0) & (j < NJ),
                                jnp.minimum(NOPS * j + k, NBLK - 1),
                                NBLK - 1), 0))


def kernel(inputs, ebd, W1, b1, W2, b2):
    idx = inputs.astype(jnp.int32)
    ebdt = ebd.T
    return pl.pallas_call(
        _mlp_body,
        grid=(2, NJ),
        in_specs=[
            pl.BlockSpec(memory_space=pltpu.SMEM),
            pl.BlockSpec(memory_space=pl.ANY),
            pl.BlockSpec((HID, NCTX * EBD), lambda p, j: (0, 0)),
            pl.BlockSpec((HID,), lambda p, j: (0,)),
            *[_w2_spec(k) for k in range(NOPS)],
            pl.BlockSpec((SPAN,), lambda p, j: (jnp.where(p == 0, j, NJ - 1),)),
        ],
        out_specs=pl.BlockSpec((1, SPAN),
                               lambda p, j: (0, jnp.where(p == 0, 0, j))),
        out_shape=jax.ShapeDtypeStruct((1, VOCAB), jnp.float32),
        scratch_shapes=[
            pltpu.VMEM((NCTX * EBD, 1), jnp.float32),
            pltpu.VMEM((NCTX, EBD, 128), jnp.float32),
            pltpu.VMEM((HID, 1), jnp.float32),
            pltpu.VMEM((NROW, BV), jnp.float32),
            pltpu.SMEM((1,), jnp.float32),
            pltpu.SMEM((1,), jnp.float32),
            pltpu.SemaphoreType.DMA,
        ],
        compiler_params=pltpu.CompilerParams(disable_bounds_checks=True),
    )(idx, ebdt, W1, b1, *([W2] * NOPS), b2)


# speedup vs baseline: 2.2649x; 1.0812x over previous
import jax
import jax.numpy as jnp
from jax import lax
from jax.experimental import pallas as pl
from jax.experimental.pallas import tpu as pltpu

VOCAB = 100000
EBD = 64
NCTX = 8
HID = 128
NOPS = 5
BV = 4096
SPAN = NOPS * BV
NJ = (VOCAB + SPAN - 1) // SPAN
NBLK = (VOCAB + BV - 1) // BV
NROW = NJ * NOPS


def _mlp_body(*refs):
    (idx_ref, ebdt_ref, w1_ref, b1_ref) = refs[:4]
    w2s = refs[4:4 + NOPS]
    (b2_ref, out_ref,
     x_ref, win_ref, h_ref, logit_ref, m_ref, s_ref, sem) = refs[4 + NOPS:]
    p = pl.program_id(0)
    j = pl.program_id(1)

    @pl.when((p == 0) & (j == 0))
    def _():
        copies = []
        for i in range(NCTX):
            base = pl.multiple_of((idx_ref[i] // 128) * 128, 128)
            c = pltpu.make_async_copy(
                ebdt_ref.at[:, pl.ds(base, 128)], win_ref.at[i], sem)
            c.start()
            copies.append(c)
        for c in copies:
            c.wait()
        lane = lax.broadcasted_iota(jnp.int32, (EBD, 128), 1)
        for i in range(NCTX):
            off = idx_ref[i] % 128
            sel = jnp.where(lane == off, win_ref[i], 0.0)
            x_ref[pl.ds(i * EBD, EBD), :] = jnp.sum(sel, axis=1,
                                                    keepdims=True)
        h = lax.dot_general(w1_ref[...], x_ref[...],
                            (((1,), (0,)), ((), ())),
                            preferred_element_type=jnp.float32)
        h_ref[...] = jnp.maximum(h + b1_ref[...][:, None], 0.0)
        m_ref[0] = -jnp.inf
        s_ref[0] = 0.0

    @pl.when(p == 0)
    def _():
        for k, w2_k in enumerate(w2s):
            logits = lax.dot_general(h_ref[...], w2_k[...],
                                     (((0,), (1,)), ((), ())),
                                     preferred_element_type=jnp.float32)
            logits = logits + b2_ref[pl.ds(k * BV, BV)][None, :]
            col = (j * SPAN + k * BV
                   + lax.broadcasted_iota(jnp.int32, (1, BV), 1))
            logits = jnp.where(col < VOCAB, logits, -jnp.inf)
            logit_ref[pl.ds(j * NOPS + k, 1), :] = logits
            bm = jnp.max(logits)
            m_old = m_ref[0]
            m_new = jnp.maximum(m_old, bm)
            s_ref[0] = s_ref[0] * jnp.exp(m_old - m_new) + jnp.sum(
                jnp.exp(logits - m_new))
            m_ref[0] = m_new

    @pl.when(p == 1)
    def _():
        lse = m_ref[0] + jnp.log(s_ref[0])
        for k in range(NOPS):
            out_ref[:, k * BV:(k + 1) * BV] = (
                logit_ref[pl.ds(j * NOPS + k, 1), :] - lse)


def _w2_spec(k):
    return pl.BlockSpec(
        (BV, HID),
        lambda p, j: (jnp.where((p == 0) & (j < NJ),
                                jnp.minimum(NOPS * j + k, NBLK - 1),
                                NBLK - 1), 0))


def kernel(inputs, ebd, W1, b1, W2, b2):
    idx = inputs.astype(jnp.int32)
    ebdt = ebd.T
    return pl.pallas_call(
        _mlp_body,
        grid=(2, NJ),
        in_specs=[
            pl.BlockSpec(memory_space=pltpu.SMEM),
            pl.BlockSpec(memory_space=pl.ANY),
            pl.BlockSpec((HID, NCTX * EBD), lambda p, j: (0, 0)),
            pl.BlockSpec((HID,), lambda p, j: (0,)),
            *[_w2_spec(k) for k in range(NOPS)],
            pl.BlockSpec((SPAN,), lambda p, j: (jnp.where(p == 0, j, NJ - 1),)),
        ],
        out_specs=pl.BlockSpec((1, SPAN),
                               lambda p, j: (0, jnp.where(p == 0, 0, j))),
        out_shape=jax.ShapeDtypeStruct((1, VOCAB), jnp.float32),
        scratch_shapes=[
            pltpu.VMEM((NCTX * EBD, 1), jnp.float32),
            pltpu.VMEM((NCTX, EBD, 128), jnp.float32),
            pltpu.VMEM((HID, 1), jnp.float32),
            pltpu.VMEM((NROW, BV), jnp.float32),
            pltpu.SMEM((1,), jnp.float32),
            pltpu.SMEM((1,), jnp.float32),
            pltpu.SemaphoreType.DMA,
        ],
        compiler_params=pltpu.CompilerParams(disable_bounds_checks=True),
    )(idx, ebdt, W1, b1, *([W2] * NOPS), b2)
